# baseline (speedup 1.0000x reference)
import functools

import jax
import jax.numpy as jnp
from jax import lax
from jax.experimental import pallas as pl
from jax.experimental.pallas import tpu as pltpu

_ALPHA = 0.2


def _attn_kernel(h_ref, m_ref, W_ref, A1_ref, A2_ref, out_ref, *rest,
                 nheads, fout, concat, emit_mask, rows):
    if emit_mask:
        mask_out_ref, hi_ref, lo_ref, e1_ref, f1_ref, e2t_ref, f2t_ref = rest
    else:
        hi_ref, lo_ref, e1_ref, f1_ref, e2t_ref, f2t_ref = rest
    i = pl.program_id(0)
    ones_in_rhs = fout <= 64
    grp = 2 * fout if ones_in_rhs else fout
    n = h_ref.shape[0]

    @pl.when(i == 0)
    def _init():
        wh = jnp.dot(h_ref[...], W_ref[...],
                     preferred_element_type=jnp.float32)
        wh1 = jnp.dot(wh, A1_ref[...], preferred_element_type=jnp.float32)
        e1_ref[...] = jnp.exp(wh1)
        f1_ref[...] = jnp.exp(_ALPHA * wh1)
        wh2t = lax.dot_general(
            A2_ref[...], wh, (((0,), (1,)), ((), ())),
            preferred_element_type=jnp.float32)
        e2t_ref[...] = jnp.exp(wh2t)
        f2t_ref[...] = jnp.exp(_ALPHA * wh2t)
        whhi = wh.astype(jnp.bfloat16)
        whlo = (wh - whhi.astype(jnp.float32)).astype(jnp.bfloat16)
        if ones_in_rhs:
            pad_hi = jnp.concatenate(
                [jnp.ones((n, 1), jnp.bfloat16),
                 jnp.zeros((n, grp - fout - 1), jnp.bfloat16)], axis=1)
            pad_lo = jnp.zeros((n, grp - fout), jnp.bfloat16)
            for h in range(nheads):
                hi_ref[:, grp * h:grp * h + fout] = (
                    whhi[:, fout * h:fout * (h + 1)])
                hi_ref[:, grp * h + fout:grp * (h + 1)] = pad_hi
                lo_ref[:, grp * h:grp * h + fout] = (
                    whlo[:, fout * h:fout * (h + 1)])
                lo_ref[:, grp * h + fout:grp * (h + 1)] = pad_lo
        else:
            hi_ref[...] = whhi
            lo_ref[...] = whlo

    if emit_mask:
        maskb = m_ref[...] > 0
        mask_out_ref[...] = maskb.astype(jnp.int8)
        maskf = maskb.astype(jnp.float32)
    else:
        maskf = m_ref[...].astype(jnp.float32)

    e1 = e1_ref[pl.ds(i * rows, rows), :]
    f1 = f1_ref[pl.ds(i * rows, rows), :]
    for h in range(nheads):
        a = e1[:, h:h + 1] * e2t_ref[h:h + 1, :]
        b = f1[:, h:h + 1] * f2t_ref[h:h + 1, :]
        pm = jnp.maximum(a, b) * maskf
        pb = pm.astype(jnp.bfloat16)
        acc = (jnp.dot(pb, hi_ref[:, grp * h:grp * (h + 1)],
                       preferred_element_type=jnp.float32)
               + jnp.dot(pb, lo_ref[:, grp * h:grp * (h + 1)],
                         preferred_element_type=jnp.float32))
        if ones_in_rhs:
            o = acc[:, :fout]
            s = acc[:, fout:fout + 1]
        else:
            o = acc
            s = jnp.sum(pm, axis=1, keepdims=True)
        o = o / s
        if concat:
            o = jnp.where(o > 0, o, jnp.exp(o) - 1.0)
        out_ref[:, h * fout:(h + 1) * fout] = o


def _gat_layer(hin, maskin, Wcat, A1, A2, nheads, fout, concat, emit_mask,
               rows=256):
    n, fin = hin.shape
    hf = nheads * fout
    grp = 2 * fout if fout <= 64 else fout
    kern = functools.partial(_attn_kernel, nheads=nheads, fout=fout,
                             concat=concat, emit_mask=emit_mask, rows=rows)
    in_specs = [
        pl.BlockSpec((n, fin), lambda i: (0, 0)),
        pl.BlockSpec((rows, n), lambda i: (i, 0)),
        pl.BlockSpec(Wcat.shape, lambda i: (0, 0)),
        pl.BlockSpec(A1.shape, lambda i: (0, 0)),
        pl.BlockSpec(A2.shape, lambda i: (0, 0)),
    ]
    out_shape = [jax.ShapeDtypeStruct((n, hf), jnp.float32)]
    out_specs = [pl.BlockSpec((rows, hf), lambda i: (i, 0))]
    if emit_mask:
        out_shape.append(jax.ShapeDtypeStruct((n, n), jnp.int8))
        out_specs.append(pl.BlockSpec((rows, n), lambda i: (i, 0)))
    return pl.pallas_call(
        kern,
        grid=(n // rows,),
        in_specs=in_specs,
        out_specs=out_specs,
        out_shape=out_shape,
        scratch_shapes=[
            pltpu.VMEM((n, grp * nheads), jnp.bfloat16),
            pltpu.VMEM((n, grp * nheads), jnp.bfloat16),
            pltpu.VMEM((n, nheads), jnp.float32),
            pltpu.VMEM((n, nheads), jnp.float32),
            pltpu.VMEM((nheads, n), jnp.float32),
            pltpu.VMEM((nheads, n), jnp.float32),
        ],
    )(hin, maskin, Wcat, A1, A2)


def kernel(x, adj, W_heads, a_heads, W_mid, a_mid, W_out, a_out):
    H, fin, F = W_heads.shape
    Wcat = jnp.transpose(W_heads, (1, 0, 2)).reshape(fin, H * F)
    a1 = a_heads[:, :F, 0]
    a2 = a_heads[:, F:, 0]
    eye = jnp.eye(H, dtype=jnp.float32)
    A1 = (a1[:, :, None] * eye[:, None, :]).reshape(H * F, H)
    A2 = (a2[:, :, None] * eye[:, None, :]).reshape(H * F, H)

    h1, mask8 = _gat_layer(x, adj, Wcat, A1, A2, H, F, True, True)

    f1 = W_mid.shape[1]
    (h2,) = _gat_layer(h1, mask8, W_mid, a_mid[:f1], a_mid[f1:],
                       1, f1, False, False)
    f2 = W_out.shape[1]
    (out,) = _gat_layer(h2, mask8, W_out, a_out[:f2], a_out[f2:],
                        1, f2, False, False)
    return out


# speedup vs baseline: 1.8779x; 1.8779x over previous
import functools

import jax
import jax.numpy as jnp
from jax import lax
from jax.experimental import pallas as pl
from jax.experimental.pallas import tpu as pltpu

_ALPHA = 0.2


def _attn_kernel(h_ref, m_ref, W_ref, A1_ref, A2_ref, out_ref, *rest,
                 nheads, fout, concat, emit_mask, rows):
    if emit_mask:
        mask_out_ref, aug_ref, e1_ref, f1_ref, e2t_ref, f2t_ref = rest
    else:
        aug_ref, e1_ref, f1_ref, e2t_ref, f2t_ref = rest
    i = pl.program_id(0)
    grp = 2 * fout
    n = h_ref.shape[0]

    @pl.when(i == 0)
    def _init():
        wh = jnp.dot(h_ref[...], W_ref[...],
                     preferred_element_type=jnp.float32)
        wh1 = jnp.dot(wh, A1_ref[...], preferred_element_type=jnp.float32)
        e1_ref[...] = jnp.exp(wh1).astype(jnp.bfloat16)
        f1_ref[...] = jnp.exp(_ALPHA * wh1).astype(jnp.bfloat16)
        wh2t = lax.dot_general(
            A2_ref[...], wh, (((0,), (1,)), ((), ())),
            preferred_element_type=jnp.float32)
        e2t_ref[...] = jnp.exp(wh2t).astype(jnp.bfloat16)
        f2t_ref[...] = jnp.exp(_ALPHA * wh2t).astype(jnp.bfloat16)
        pad = jnp.concatenate(
            [jnp.ones((n, 1), jnp.bfloat16),
             jnp.zeros((n, grp - fout - 1), jnp.bfloat16)], axis=1)
        whb = wh.astype(jnp.bfloat16)
        for h in range(nheads):
            aug_ref[:, grp * h:grp * h + fout] = (
                whb[:, fout * h:fout * (h + 1)])
            aug_ref[:, grp * h + fout:grp * (h + 1)] = pad

    if emit_mask:
        maskb = m_ref[...] > 0
        mask_out_ref[...] = maskb.astype(jnp.int8)
        maskf = maskb.astype(jnp.bfloat16)
    else:
        maskf = m_ref[...].astype(jnp.bfloat16)

    e1 = e1_ref[pl.ds(i * rows, rows), :]
    f1 = f1_ref[pl.ds(i * rows, rows), :]
    for h in range(nheads):
        a = e1[:, h:h + 1] * e2t_ref[h:h + 1, :]
        b = f1[:, h:h + 1] * f2t_ref[h:h + 1, :]
        pm = jnp.maximum(a, b) * maskf
        acc = jnp.dot(pm, aug_ref[:, grp * h:grp * (h + 1)],
                      preferred_element_type=jnp.float32)
        o = acc[:, :fout] / acc[:, fout:fout + 1]
        if concat:
            o = jnp.where(o > 0, o, jnp.exp(o) - 1.0)
        out_ref[:, h * fout:(h + 1) * fout] = o


def _gat_layer(hin, maskin, Wcat, A1, A2, nheads, fout, concat, emit_mask,
               rows=256):
    n, fin = hin.shape
    hf = nheads * fout
    grp = 2 * fout
    kern = functools.partial(_attn_kernel, nheads=nheads, fout=fout,
                             concat=concat, emit_mask=emit_mask, rows=rows)
    in_specs = [
        pl.BlockSpec((n, fin), lambda i: (0, 0)),
        pl.BlockSpec((rows, n), lambda i: (i, 0)),
        pl.BlockSpec(Wcat.shape, lambda i: (0, 0)),
        pl.BlockSpec(A1.shape, lambda i: (0, 0)),
        pl.BlockSpec(A2.shape, lambda i: (0, 0)),
    ]
    out_shape = [jax.ShapeDtypeStruct((n, hf), jnp.float32)]
    out_specs = [pl.BlockSpec((rows, hf), lambda i: (i, 0))]
    if emit_mask:
        out_shape.append(jax.ShapeDtypeStruct((n, n), jnp.int8))
        out_specs.append(pl.BlockSpec((rows, n), lambda i: (i, 0)))
    return pl.pallas_call(
        kern,
        grid=(n // rows,),
        in_specs=in_specs,
        out_specs=out_specs,
        out_shape=out_shape,
        scratch_shapes=[
            pltpu.VMEM((n, grp * nheads), jnp.bfloat16),
            pltpu.VMEM((n, nheads), jnp.bfloat16),
            pltpu.VMEM((n, nheads), jnp.bfloat16),
            pltpu.VMEM((nheads, n), jnp.bfloat16),
            pltpu.VMEM((nheads, n), jnp.bfloat16),
        ],
    )(hin, maskin, Wcat, A1, A2)


def kernel(x, adj, W_heads, a_heads, W_mid, a_mid, W_out, a_out):
    H, fin, F = W_heads.shape
    Wcat = jnp.transpose(W_heads, (1, 0, 2)).reshape(fin, H * F)
    a1 = a_heads[:, :F, 0]
    a2 = a_heads[:, F:, 0]
    eye = jnp.eye(H, dtype=jnp.float32)
    A1 = (a1[:, :, None] * eye[:, None, :]).reshape(H * F, H)
    A2 = (a2[:, :, None] * eye[:, None, :]).reshape(H * F, H)

    h1, mask8 = _gat_layer(x, adj, Wcat, A1, A2, H, F, True, True)

    f1 = W_mid.shape[1]
    (h2,) = _gat_layer(h1, mask8, W_mid, a_mid[:f1], a_mid[f1:],
                       1, f1, False, False)
    f2 = W_out.shape[1]
    (out,) = _gat_layer(h2, mask8, W_out, a_out[:f2], a_out[f2:],
                        1, f2, False, False)
    return out


# speedup vs baseline: 2.1530x; 1.1465x over previous
import functools

import jax
import jax.numpy as jnp
from jax import lax
from jax.experimental import pallas as pl
from jax.experimental.pallas import tpu as pltpu

_ALPHA = 0.2


def _attn_kernel(h_ref, m_ref, W_ref, A1_ref, A2_ref, out_ref, *rest,
                 nheads, fout, concat, emit_mask, rows):
    if emit_mask:
        mask_out_ref, aug_ref, e1_ref, f1_ref, e2t_ref, f2t_ref = rest
    else:
        aug_ref, e1_ref, f1_ref, e2t_ref, f2t_ref = rest
    i = pl.program_id(0)
    grp = 128 if fout <= 64 else 2 * fout
    n = h_ref.shape[0]

    @pl.when(i == 0)
    def _init():
        wh = jnp.dot(h_ref[...], W_ref[...],
                     preferred_element_type=jnp.float32)
        wh1 = jnp.dot(wh, A1_ref[...], preferred_element_type=jnp.float32)
        e1_ref[...] = jnp.exp(wh1).astype(jnp.bfloat16)
        f1_ref[...] = jnp.exp(_ALPHA * wh1).astype(jnp.bfloat16)
        wh2t = lax.dot_general(
            A2_ref[...], wh, (((0,), (1,)), ((), ())),
            preferred_element_type=jnp.float32)
        e2t_ref[...] = jnp.exp(wh2t).astype(jnp.bfloat16)
        f2t_ref[...] = jnp.exp(_ALPHA * wh2t).astype(jnp.bfloat16)
        pad = jnp.concatenate(
            [jnp.ones((n, 1), jnp.bfloat16),
             jnp.zeros((n, grp - fout - 1), jnp.bfloat16)], axis=1)
        whb = wh.astype(jnp.bfloat16)
        for h in range(nheads):
            aug_ref[:, grp * h:grp * h + fout] = (
                whb[:, fout * h:fout * (h + 1)])
            aug_ref[:, grp * h + fout:grp * (h + 1)] = pad

    if emit_mask:
        maskb = m_ref[...] > 0
        mask_out_ref[...] = maskb.astype(jnp.int8)
        maskf = maskb.astype(jnp.bfloat16)
    else:
        maskf = m_ref[...].astype(jnp.bfloat16)

    e1 = e1_ref[pl.ds(i * rows, rows), :]
    f1 = f1_ref[pl.ds(i * rows, rows), :]
    for h in range(nheads):
        a = e1[:, h:h + 1] * e2t_ref[h:h + 1, :]
        b = f1[:, h:h + 1] * f2t_ref[h:h + 1, :]
        pm = jnp.maximum(a, b) * maskf
        acc = jnp.dot(pm, aug_ref[:, grp * h:grp * (h + 1)],
                      preferred_element_type=jnp.float32)
        o = acc[:, :fout] / acc[:, fout:fout + 1]
        if concat:
            o = jnp.where(o > 0, o, jnp.exp(o) - 1.0)
        out_ref[:, h * fout:(h + 1) * fout] = o


def _gat_layer(hin, maskin, Wcat, A1, A2, nheads, fout, concat, emit_mask,
               rows=512):
    n, fin = hin.shape
    hf = nheads * fout
    grp = 128 if fout <= 64 else 2 * fout
    kern = functools.partial(_attn_kernel, nheads=nheads, fout=fout,
                             concat=concat, emit_mask=emit_mask, rows=rows)
    in_specs = [
        pl.BlockSpec((n, fin), lambda i: (0, 0)),
        pl.BlockSpec((rows, n), lambda i: (i, 0)),
        pl.BlockSpec(Wcat.shape, lambda i: (0, 0)),
        pl.BlockSpec(A1.shape, lambda i: (0, 0)),
        pl.BlockSpec(A2.shape, lambda i: (0, 0)),
    ]
    out_shape = [jax.ShapeDtypeStruct((n, hf), jnp.float32)]
    out_specs = [pl.BlockSpec((rows, hf), lambda i: (i, 0))]
    if emit_mask:
        out_shape.append(jax.ShapeDtypeStruct((n, n), jnp.int8))
        out_specs.append(pl.BlockSpec((rows, n), lambda i: (i, 0)))
    return pl.pallas_call(
        kern,
        grid=(n // rows,),
        in_specs=in_specs,
        out_specs=out_specs,
        out_shape=out_shape,
        scratch_shapes=[
            pltpu.VMEM((n, grp * nheads), jnp.bfloat16),
            pltpu.VMEM((n, nheads), jnp.bfloat16),
            pltpu.VMEM((n, nheads), jnp.bfloat16),
            pltpu.VMEM((nheads, n), jnp.bfloat16),
            pltpu.VMEM((nheads, n), jnp.bfloat16),
        ],
    )(hin, maskin, Wcat, A1, A2)


def kernel(x, adj, W_heads, a_heads, W_mid, a_mid, W_out, a_out):
    H, fin, F = W_heads.shape
    Wcat = jnp.transpose(W_heads, (1, 0, 2)).reshape(fin, H * F)
    a1 = a_heads[:, :F, 0]
    a2 = a_heads[:, F:, 0]
    eye = jnp.eye(H, dtype=jnp.float32)
    A1 = (a1[:, :, None] * eye[:, None, :]).reshape(H * F, H)
    A2 = (a2[:, :, None] * eye[:, None, :]).reshape(H * F, H)

    h1, mask8 = _gat_layer(x, adj, Wcat, A1, A2, H, F, True, True)

    f1 = W_mid.shape[1]
    (h2,) = _gat_layer(h1, mask8, W_mid, a_mid[:f1], a_mid[f1:],
                       1, f1, False, False)
    f2 = W_out.shape[1]
    (out,) = _gat_layer(h2, mask8, W_out, a_out[:f2], a_out[f2:],
                        1, f2, False, False)
    return out
